# baseline (speedup 1.0000x reference)
import functools

import jax
import jax.numpy as jnp
from jax import lax
from jax.experimental import pallas as pl
from jax.experimental.pallas import tpu as pltpu
from jax.experimental.pallas import tpu_sc as plsc

EMBED = 64
LANES = 16

_info = plsc.get_sparse_core_info()
_NC, _NS = _info.num_cores, _info.num_subcores
_NW = _NC * _NS

CHUNK = 128


def _embedding_gather(Wp, xt, H, B):
    J = xt.shape[0]
    n_chunks = J // CHUNK
    per_w = n_chunks // _NW
    assert per_w * _NW == n_chunks and per_w % 2 == 0
    j_per_w = per_w * CHUNK
    bt_per_h = B // CHUNK

    mesh = plsc.VectorSubcoreMesh(core_axis_name="c", subcore_axis_name="s")

    @functools.partial(
        pl.kernel,
        mesh=mesh,
        out_type=jax.ShapeDtypeStruct((H, EMBED, B), jnp.float32),
        scratch_types=[
            pltpu.VMEM((j_per_w,), jnp.int32),
            pltpu.VMEM((2, CHUNK), jnp.int32),
            pltpu.VMEM((2, CHUNK, 128), jnp.float32),
            pltpu.VMEM((2, EMBED, CHUNK), jnp.float32),
            [pltpu.SemaphoreType.DMA] * 2,
            [pltpu.SemaphoreType.DMA] * 2,
        ],
        compiler_params=pltpu.CompilerParams(
            use_tc_tiling_on_sc=True, needs_layout_passes=False),
    )
    def k(wp_hbm, xt_hbm, out_hbm, idx_v, kbuf, gbuf, obuf, gsems, osems):
        wid = lax.axis_index("s") * _NC + lax.axis_index("c")
        g0 = wid * per_w

        pltpu.sync_copy(xt_hbm.at[pl.ds(wid * j_per_w, j_per_w)], idx_v)

        iota = lax.iota(jnp.int32, LANES)

        def fill_kbuf(t, bi):
            for q in range(CHUNK // LANES):
                xv = idx_v[pl.ds(t * CHUNK + q * LANES, LANES)]
                kbuf[bi, pl.ds(q * LANES, LANES)] = lax.shift_right_logical(
                    xv, 1)

        def start_gather(bi):
            pltpu.async_copy(wp_hbm.at[kbuf.at[bi]], gbuf.at[bi], gsems[bi])

        def wait_gather(bi):
            pltpu.make_async_copy(
                wp_hbm.at[kbuf.at[bi]], gbuf.at[bi], gsems[bi]).wait()

        def out_ref(t):
            g = g0 + t
            h = g // bt_per_h
            bt = g % bt_per_h
            return out_hbm.at[h, pl.ds(0, EMBED), pl.ds(bt * CHUNK, CHUNK)]

        def start_store(t, bi):
            pltpu.async_copy(obuf.at[bi], out_ref(t), osems[bi])

        def wait_store(t, bi):
            pltpu.make_async_copy(obuf.at[bi], out_ref(t), osems[bi]).wait()

        def transpose_chunk(t, bi):
            g2 = gbuf.at[bi]
            for q in range(CHUNK // LANES):
                xv = idx_v[pl.ds(t * CHUNK + q * LANES, LANES)]
                colbase = lax.shift_left(lax.bitwise_and(xv, 1), 6)
                row = iota + (q * LANES)
                for c in range(EMBED):
                    vals = plsc.load_gather(g2, [row, colbase + c])
                    obuf[bi, c, pl.ds(q * LANES, LANES)] = vals

        fill_kbuf(0, 0)
        start_gather(0)

        def outer(p, carry):
            for bi in range(2):
                t = p * 2 + bi
                nxt = 1 - bi

                @pl.when(t + 1 < per_w)
                def _():
                    fill_kbuf(t + 1, nxt)
                    start_gather(nxt)

                @pl.when(t >= 2)
                def _():
                    wait_store(t - 2, bi)

                wait_gather(bi)
                transpose_chunk(t, bi)
                start_store(t, bi)
            return carry

        lax.fori_loop(0, per_w // 2, outer, 0)
        wait_store(per_w - 2, 0)
        wait_store(per_w - 1, 1)

    return k(Wp, xt)


def kernel(x, W_embed):
    H_, B_ = x.shape[1], x.shape[0]
    Wp = W_embed.reshape(500000, 128)
    xt = x.T.reshape(-1).astype(jnp.int32)
    out_t = _embedding_gather(Wp, xt, H_, B_)
    return out_t.transpose(2, 0, 1)


# speedup vs baseline: 1.9978x; 1.9978x over previous
import functools

import jax
import jax.numpy as jnp
from jax import lax
from jax.experimental import pallas as pl
from jax.experimental.pallas import tpu as pltpu
from jax.experimental.pallas import tpu_sc as plsc

EMBED = 64
LANES = 16

_info = plsc.get_sparse_core_info()
_NC, _NS = _info.num_cores, _info.num_subcores
_NW = _NC * _NS

CHUNK = 128


def _embedding_gather(Wp, xt, H, B):
    J = xt.shape[0]
    n_chunks = J // CHUNK
    per_w = n_chunks // _NW
    assert per_w * _NW == n_chunks and per_w % 2 == 0
    j_per_w = per_w * CHUNK
    bt_per_h = B // CHUNK

    mesh = plsc.VectorSubcoreMesh(core_axis_name="c", subcore_axis_name="s")

    @functools.partial(
        pl.kernel,
        mesh=mesh,
        out_type=jax.ShapeDtypeStruct((H, EMBED, B), jnp.float32),
        scratch_types=[
            pltpu.VMEM((j_per_w,), jnp.int32),
            pltpu.VMEM((2, CHUNK), jnp.int32),
            pltpu.VMEM((2, CHUNK, 128), jnp.float32),
            pltpu.VMEM((2, EMBED, CHUNK), jnp.float32),
            [pltpu.SemaphoreType.DMA] * 2,
            [pltpu.SemaphoreType.DMA] * 2,
        ],
        compiler_params=pltpu.CompilerParams(
            use_tc_tiling_on_sc=True, needs_layout_passes=False),
    )
    def k(wp_hbm, xt_hbm, out_hbm, idx_v, kbuf, gbuf, obuf, gsems, osems):
        wid = lax.axis_index("s") * _NC + lax.axis_index("c")
        g0 = wid * per_w

        pltpu.sync_copy(xt_hbm.at[pl.ds(wid * j_per_w, j_per_w)], idx_v)

        iota = lax.iota(jnp.int32, LANES)

        def fill_kbuf(t, bi):
            for q in range(CHUNK // LANES):
                xv = idx_v[pl.ds(t * CHUNK + q * LANES, LANES)]
                kbuf[bi, pl.ds(q * LANES, LANES)] = lax.shift_right_logical(
                    xv, 1)

        def start_gather(bi):
            pltpu.async_copy(wp_hbm.at[kbuf.at[bi]], gbuf.at[bi], gsems[bi])

        def wait_gather(bi):
            pltpu.make_async_copy(
                wp_hbm.at[kbuf.at[bi]], gbuf.at[bi], gsems[bi]).wait()

        def out_ref(t):
            g = g0 + t
            h = g // bt_per_h
            bt = g % bt_per_h
            return out_hbm.at[h, pl.ds(0, EMBED), pl.ds(bt * CHUNK, CHUNK)]

        def start_store(t, bi):
            pltpu.async_copy(obuf.at[bi], out_ref(t), osems[bi])

        def wait_store(t, bi):
            pltpu.make_async_copy(obuf.at[bi], out_ref(t), osems[bi]).wait()

        def transpose_chunk(t, bi):
            g2 = gbuf.at[bi]
            ob = obuf.at[bi]
            colbases = []
            for q in range(CHUNK // LANES):
                xv = idx_v[pl.ds(t * CHUNK + q * LANES, LANES)]
                colbases.append(lax.shift_left(lax.bitwise_and(xv, 1), 6))

            def cgroup(cg, carry):
                for dc in range(16):
                    cvec = lax.bitwise_and(iota + (cg * 16 + dc), 63)
                    for q in range(CHUNK // LANES):
                        rowj = iota + (q * LANES)
                        vals = plsc.load_gather(
                            g2, [rowj, colbases[q] + cvec])
                        plsc.store_scatter(ob, [cvec, rowj], vals)
                return carry

            lax.fori_loop(0, EMBED // 16, cgroup, 0)

        fill_kbuf(0, 0)
        start_gather(0)

        def outer(p, carry):
            for bi in range(2):
                t = p * 2 + bi
                nxt = 1 - bi

                @pl.when(t + 1 < per_w)
                def _():
                    fill_kbuf(t + 1, nxt)
                    start_gather(nxt)

                @pl.when(t >= 2)
                def _():
                    wait_store(t - 2, bi)

                wait_gather(bi)
                transpose_chunk(t, bi)
                start_store(t, bi)
            return carry

        lax.fori_loop(0, per_w // 2, outer, 0)
        wait_store(per_w - 2, 0)
        wait_store(per_w - 1, 1)

    return k(Wp, xt)


def kernel(x, W_embed):
    H_, B_ = x.shape[1], x.shape[0]
    Wp = W_embed.reshape(500000, 128)
    xt = x.T.reshape(-1).astype(jnp.int32)
    out_t = _embedding_gather(Wp, xt, H_, B_)
    return out_t.transpose(2, 0, 1)


# speedup vs baseline: 2.3521x; 1.1773x over previous
import functools

import jax
import jax.numpy as jnp
from jax import lax
from jax.experimental import pallas as pl
from jax.experimental.pallas import tpu as pltpu
from jax.experimental.pallas import tpu_sc as plsc

EMBED = 64
LANES = 16
VOCAB = 1000000
VOCAB_PAD = 1000064
NBLK = VOCAB_PAD // 128

_info = plsc.get_sparse_core_info()
_NC, _NS = _info.num_cores, _info.num_subcores
_NW = _NC * _NS

CHUNK = 128

_CPARAMS = pltpu.CompilerParams(
    use_tc_tiling_on_sc=True, needs_layout_passes=False)

_MAIN_BLOCKS = 244


def _table_transpose(Wt, Wtail):
    mesh = plsc.VectorSubcoreMesh(core_axis_name="c", subcore_axis_name="s")

    @functools.partial(
        pl.kernel,
        mesh=mesh,
        out_type=jax.ShapeDtypeStruct((VOCAB_PAD // 2, 128), jnp.float32),
        scratch_types=[
            pltpu.VMEM((2, EMBED, 128), jnp.float32),
            pltpu.VMEM((2, EMBED, 128), jnp.float32),
            pltpu.VMEM((EMBED, EMBED), jnp.float32),
            pltpu.VMEM((32, 128), jnp.float32),
            [pltpu.SemaphoreType.DMA] * 2,
            [pltpu.SemaphoreType.DMA] * 2,
        ],
        compiler_params=_CPARAMS,
    )
    def k(wt_hbm, wtail_hbm, out_hbm, ibuf, obuf, ibuf64, obuf64,
          isems, osems):
        wid = lax.axis_index("s") * _NC + lax.axis_index("c")

        iota = lax.iota(jnp.int32, LANES)

        def blk(m):
            return wid + _NW * m

        def in_view(m, bi):
            return (wt_hbm.at[pl.ds(0, EMBED), pl.ds(blk(m) * 128, 128)],
                    ibuf.at[bi], isems[bi])

        def out_view(m, bi):
            return (obuf.at[bi],
                    out_hbm.at[pl.ds(blk(m) * EMBED, EMBED)],
                    osems[bi])

        def transpose(src, dst, nr):
            def cgroup(cg, carry):
                for dc in range(LANES):
                    cvec = lax.bitwise_and(iota + (cg * LANES + dc), 63)
                    for q in range(nr // LANES):
                        rowj = iota + (q * LANES)
                        flat = rowj * EMBED + cvec
                        vals = plsc.load_gather(src, [cvec, rowj])
                        plsc.store_scatter(
                            dst,
                            [lax.shift_right_logical(flat, 7),
                             lax.bitwise_and(flat, 127)],
                            vals)
                return carry

            lax.fori_loop(0, EMBED // LANES, cgroup, 0)

        pltpu.async_copy(*in_view(0, 0))
        pltpu.async_copy(*in_view(1, 1))

        def body(p, carry):
            for bi in range(2):
                m = p * 2 + bi

                pltpu.make_async_copy(*in_view(m, bi)).wait()

                @pl.when(m >= 2)
                def _():
                    pltpu.make_async_copy(*out_view(m - 2, bi)).wait()

                transpose(ibuf.at[bi], obuf.at[bi], 128)
                pltpu.async_copy(*out_view(m, bi))

                @pl.when(m + 2 < _MAIN_BLOCKS)
                def _():
                    pltpu.async_copy(*in_view(m + 2, bi))
            return carry

        lax.fori_loop(0, _MAIN_BLOCKS // 2, body, 0)
        pltpu.make_async_copy(*out_view(_MAIN_BLOCKS - 2, 0)).wait()
        pltpu.make_async_copy(*out_view(_MAIN_BLOCKS - 1, 1)).wait()

        @pl.when(wid < NBLK - 1 - _MAIN_BLOCKS * _NW)
        def _():
            b = _MAIN_BLOCKS * _NW + wid
            pltpu.sync_copy(
                wt_hbm.at[pl.ds(0, EMBED), pl.ds(b * 128, 128)],
                ibuf.at[0])
            transpose(ibuf.at[0], obuf.at[0], 128)
            pltpu.sync_copy(obuf.at[0], out_hbm.at[pl.ds(b * EMBED, EMBED)])

        @pl.when(wid == 4)
        def _():
            pltpu.sync_copy(wtail_hbm, ibuf64)
            transpose(ibuf64, obuf64, EMBED)
            pltpu.sync_copy(
                obuf64, out_hbm.at[pl.ds((NBLK - 1) * EMBED, 32)])

    return k(Wt, Wtail)


def _embedding_gather(Wp, xt, H, B):
    J = xt.shape[0]
    n_chunks = J // CHUNK
    per_w = n_chunks // _NW
    assert per_w * _NW == n_chunks and per_w % 2 == 0
    j_per_w = per_w * CHUNK
    bt_per_h = B // CHUNK

    mesh = plsc.VectorSubcoreMesh(core_axis_name="c", subcore_axis_name="s")

    @functools.partial(
        pl.kernel,
        mesh=mesh,
        out_type=jax.ShapeDtypeStruct((H, EMBED, B), jnp.float32),
        scratch_types=[
            pltpu.VMEM((j_per_w,), jnp.int32),
            pltpu.VMEM((2, CHUNK), jnp.int32),
            pltpu.VMEM((2, CHUNK, 128), jnp.float32),
            pltpu.VMEM((2, EMBED, CHUNK), jnp.float32),
            [pltpu.SemaphoreType.DMA] * 2,
            [pltpu.SemaphoreType.DMA] * 2,
        ],
        compiler_params=_CPARAMS,
    )
    def k(wp_hbm, xt_hbm, out_hbm, idx_v, kbuf, gbuf, obuf, gsems, osems):
        wid = lax.axis_index("s") * _NC + lax.axis_index("c")
        g0 = wid * per_w

        pltpu.sync_copy(xt_hbm.at[pl.ds(wid * j_per_w, j_per_w)], idx_v)

        iota = lax.iota(jnp.int32, LANES)

        def fill_kbuf(t, bi):
            for q in range(CHUNK // LANES):
                xv = idx_v[pl.ds(t * CHUNK + q * LANES, LANES)]
                kbuf[bi, pl.ds(q * LANES, LANES)] = lax.shift_right_logical(
                    xv, 1)

        def start_gather(bi):
            pltpu.async_copy(wp_hbm.at[kbuf.at[bi]], gbuf.at[bi], gsems[bi])

        def wait_gather(bi):
            pltpu.make_async_copy(
                wp_hbm.at[kbuf.at[bi]], gbuf.at[bi], gsems[bi]).wait()

        def out_ref(t):
            g = g0 + t
            h = g // bt_per_h
            bt = g % bt_per_h
            return out_hbm.at[h, pl.ds(0, EMBED), pl.ds(bt * CHUNK, CHUNK)]

        def start_store(t, bi):
            pltpu.async_copy(obuf.at[bi], out_ref(t), osems[bi])

        def wait_store(t, bi):
            pltpu.make_async_copy(obuf.at[bi], out_ref(t), osems[bi]).wait()

        def transpose_chunk(t, bi):
            g2 = gbuf.at[bi]
            ob = obuf.at[bi]
            colbases = []
            for q in range(CHUNK // LANES):
                xv = idx_v[pl.ds(t * CHUNK + q * LANES, LANES)]
                colbases.append(lax.shift_left(lax.bitwise_and(xv, 1), 6))

            def cgroup(cg, carry):
                for dc in range(LANES):
                    cvec = lax.bitwise_and(iota + (cg * LANES + dc), 63)
                    for q in range(CHUNK // LANES):
                        rowj = iota + (q * LANES)
                        vals = plsc.load_gather(
                            g2, [rowj, colbases[q] + cvec])
                        plsc.store_scatter(ob, [cvec, rowj], vals)
                return carry

            lax.fori_loop(0, EMBED // LANES, cgroup, 0)

        fill_kbuf(0, 0)
        start_gather(0)

        def outer(p, carry):
            for bi in range(2):
                t = p * 2 + bi
                nxt = 1 - bi

                @pl.when(t + 1 < per_w)
                def _():
                    fill_kbuf(t + 1, nxt)
                    start_gather(nxt)

                @pl.when(t >= 2)
                def _():
                    wait_store(t - 2, bi)

                wait_gather(bi)
                transpose_chunk(t, bi)
                start_store(t, bi)
            return carry

        lax.fori_loop(0, per_w // 2, outer, 0)
        wait_store(per_w - 2, 0)
        wait_store(per_w - 1, 1)

    return k(Wp, xt)


def kernel(x, W_embed):
    B_, H_ = x.shape
    xt = x.T.reshape(-1).astype(jnp.int32)
    Wtail = W_embed[VOCAB - EMBED:, :].T
    Wp = _table_transpose(W_embed.T, Wtail)
    out_t = _embedding_gather(Wp, xt, H_, B_)
    return out_t.transpose(2, 0, 1)


# speedup vs baseline: 2.3622x; 1.0043x over previous
import functools

import jax
import jax.numpy as jnp
from jax import lax
from jax.experimental import pallas as pl
from jax.experimental.pallas import tpu as pltpu
from jax.experimental.pallas import tpu_sc as plsc

EMBED = 64
LANES = 16
VOCAB = 1000000
VOCAB_PAD = 1000064
NBLK = VOCAB_PAD // 128

_info = plsc.get_sparse_core_info()
_NC, _NS = _info.num_cores, _info.num_subcores
_NW = _NC * _NS

CHUNK = 128

_CPARAMS = pltpu.CompilerParams(
    use_tc_tiling_on_sc=True, needs_layout_passes=False)
_CPARAMS_LIN = pltpu.CompilerParams(
    use_tc_tiling_on_sc=False, needs_layout_passes=False)

_MAIN_BLOCKS = 244


def _table_transpose(Wt, Wtail):
    mesh = plsc.VectorSubcoreMesh(core_axis_name="c", subcore_axis_name="s")

    @functools.partial(
        pl.kernel,
        mesh=mesh,
        out_type=jax.ShapeDtypeStruct((VOCAB_PAD // 2, 128), jnp.float32),
        scratch_types=[
            pltpu.VMEM((2, EMBED, 128), jnp.float32),
            pltpu.VMEM((2, EMBED, 128), jnp.float32),
            pltpu.VMEM((EMBED, EMBED), jnp.float32),
            pltpu.VMEM((32, 128), jnp.float32),
            [pltpu.SemaphoreType.DMA] * 2,
            [pltpu.SemaphoreType.DMA] * 2,
        ],
        compiler_params=_CPARAMS,
    )
    def k(wt_hbm, wtail_hbm, out_hbm, ibuf, obuf, ibuf64, obuf64,
          isems, osems):
        wid = lax.axis_index("s") * _NC + lax.axis_index("c")

        iota = lax.iota(jnp.int32, LANES)

        def blk(m):
            return wid + _NW * m

        def in_view(m, bi):
            return (wt_hbm.at[pl.ds(0, EMBED), pl.ds(blk(m) * 128, 128)],
                    ibuf.at[bi], isems[bi])

        def out_view(m, bi):
            return (obuf.at[bi],
                    out_hbm.at[pl.ds(blk(m) * EMBED, EMBED)],
                    osems[bi])

        def transpose(src, dst, nr):
            def cgroup(cg, carry):
                for dc in range(LANES):
                    cvec = lax.bitwise_and(iota + (cg * LANES + dc), 63)
                    for q in range(nr // LANES):
                        rowj = iota + (q * LANES)
                        flat = rowj * EMBED + cvec
                        vals = plsc.load_gather(src, [cvec, rowj])
                        plsc.store_scatter(
                            dst,
                            [lax.shift_right_logical(flat, 7),
                             lax.bitwise_and(flat, 127)],
                            vals)
                return carry

            lax.fori_loop(0, EMBED // LANES, cgroup, 0)

        pltpu.async_copy(*in_view(0, 0))
        pltpu.async_copy(*in_view(1, 1))

        def body(p, carry):
            for bi in range(2):
                m = p * 2 + bi

                pltpu.make_async_copy(*in_view(m, bi)).wait()

                @pl.when(m >= 2)
                def _():
                    pltpu.make_async_copy(*out_view(m - 2, bi)).wait()

                transpose(ibuf.at[bi], obuf.at[bi], 128)
                pltpu.async_copy(*out_view(m, bi))

                @pl.when(m + 2 < _MAIN_BLOCKS)
                def _():
                    pltpu.async_copy(*in_view(m + 2, bi))
            return carry

        lax.fori_loop(0, _MAIN_BLOCKS // 2, body, 0)
        pltpu.make_async_copy(*out_view(_MAIN_BLOCKS - 2, 0)).wait()
        pltpu.make_async_copy(*out_view(_MAIN_BLOCKS - 1, 1)).wait()

        @pl.when(wid < NBLK - 1 - _MAIN_BLOCKS * _NW)
        def _():
            b = _MAIN_BLOCKS * _NW + wid
            pltpu.sync_copy(
                wt_hbm.at[pl.ds(0, EMBED), pl.ds(b * 128, 128)],
                ibuf.at[0])
            transpose(ibuf.at[0], obuf.at[0], 128)
            pltpu.sync_copy(obuf.at[0], out_hbm.at[pl.ds(b * EMBED, EMBED)])

        @pl.when(wid == 4)
        def _():
            pltpu.sync_copy(wtail_hbm, ibuf64)
            transpose(ibuf64, obuf64, EMBED)
            pltpu.sync_copy(
                obuf64, out_hbm.at[pl.ds((NBLK - 1) * EMBED, 32)])

    return k(Wt, Wtail)


def _embedding_gather(Wr, xt, H, B):
    J = xt.shape[0]
    n_chunks = J // CHUNK
    per_w = n_chunks // _NW
    assert per_w * _NW == n_chunks and per_w % 2 == 0
    j_per_w = per_w * CHUNK
    bt_per_h = B // CHUNK

    mesh = plsc.VectorSubcoreMesh(core_axis_name="c", subcore_axis_name="s")

    @functools.partial(
        pl.kernel,
        mesh=mesh,
        out_type=jax.ShapeDtypeStruct((H, 8, bt_per_h, 8, CHUNK),
                                      jnp.float32),
        scratch_types=[
            pltpu.VMEM((j_per_w,), jnp.int32),
            pltpu.VMEM((2, CHUNK, EMBED), jnp.float32),
            pltpu.VMEM((2, 8, 8, CHUNK), jnp.float32),
            [pltpu.SemaphoreType.DMA] * 2,
            [pltpu.SemaphoreType.DMA] * 2,
        ],
        compiler_params=_CPARAMS_LIN,
    )
    def k(wr_hbm, xt_hbm, out_hbm, idx_v, gbuf, obuf, gsems, osems):
        wid = lax.axis_index("s") * _NC + lax.axis_index("c")
        g0 = wid * per_w

        pltpu.sync_copy(xt_hbm.at[pl.ds(wid * j_per_w, j_per_w)], idx_v)

        iota = lax.iota(jnp.int32, LANES)

        def idx_ref(t):
            return idx_v.at[pl.ds(t * CHUNK, CHUNK)]

        def start_gather(t, bi):
            pltpu.async_copy(wr_hbm.at[idx_ref(t)], gbuf.at[bi], gsems[bi])

        def wait_gather(t, bi):
            pltpu.make_async_copy(
                wr_hbm.at[idx_ref(t)], gbuf.at[bi], gsems[bi]).wait()

        def store_views(t, bi):
            g = g0 + t
            h = g // bt_per_h
            bt = g % bt_per_h
            return [(obuf.at[bi, ch], out_hbm.at[h, ch, bt], osems[bi])
                    for ch in range(8)]

        def start_store(t, bi):
            for v in store_views(t, bi):
                pltpu.async_copy(*v)

        def wait_store(t, bi):
            for v in store_views(t, bi):
                pltpu.make_async_copy(*v).wait()

        def transpose_chunk(bi):
            g2 = gbuf.at[bi]
            ob = obuf.at[bi]

            def cgroup(cg, carry):
                for dc in range(LANES):
                    cvec = lax.bitwise_and(iota + (cg * LANES + dc), 63)
                    ch = lax.shift_right_logical(cvec, 3)
                    cl = lax.bitwise_and(cvec, 7)
                    for q in range(CHUNK // LANES):
                        rowj = iota + (q * LANES)
                        vals = plsc.load_gather(g2, [rowj, cvec])
                        plsc.store_scatter(ob, [ch, cl, rowj], vals)
                return carry

            lax.fori_loop(0, EMBED // LANES, cgroup, 0)

        start_gather(0, 0)

        def outer(p, carry):
            for bi in range(2):
                t = p * 2 + bi

                @pl.when(t + 1 < per_w)
                def _():
                    start_gather(t + 1, 1 - bi)

                @pl.when(t >= 2)
                def _():
                    wait_store(t - 2, bi)

                wait_gather(t, bi)
                transpose_chunk(bi)
                start_store(t, bi)
            return carry

        lax.fori_loop(0, per_w // 2, outer, 0)
        wait_store(per_w - 2, 0)
        wait_store(per_w - 1, 1)

    return k(Wr, xt)


def kernel(x, W_embed):
    B_, H_ = x.shape
    xt = x.T.reshape(-1).astype(jnp.int32)
    Wtail = W_embed[VOCAB - EMBED:, :].T
    Wr = _table_transpose(W_embed.T, Wtail).reshape(VOCAB_PAD, EMBED)
    out5 = _embedding_gather(Wr, xt, H_, B_)
    return out5.transpose(2, 4, 0, 1, 3).reshape(B_, H_, EMBED)
